# baseline (speedup 1.0000x reference)
import functools

import jax
import jax.numpy as jnp
from jax import lax
from jax.experimental import pallas as pl
from jax.experimental.pallas import tpu as pltpu
from jax.experimental.pallas import tpu_sc as plsc

M, C, D = 65536, 8, 64
B, S, K = 8, 2048, 4
N = B * S
L = 16
ND = D // L

NC, NS = 2, 16
NW = NC * NS
TOK_PER_W = N // NW
T = 32
CHUNKS = TOK_PER_W // T

BM = 4096


def _mean_body(st_ref, out_ref):
    acc = st_ref[:, 0, :]
    for c in range(1, C):
        acc = acc + st_ref[:, c, :]
    acc = acc * (1.0 / C)
    out_ref[:, 0:D] = acc
    out_ref[:, D:2 * D] = acc


def _sc_read(idx_hbm, q_hbm, mn_hbm, out_hbm,
             idx0, idx1, rows0, rows1, q0, q1, o0, o1, sem0, sem1):
    wid = lax.axis_index("s") * NC + lax.axis_index("c")
    tok0 = wid * TOK_PER_W
    bufs = ((idx0, rows0, q0, o0, sem0), (idx1, rows1, q1, o1, sem1))

    def issue(ch, b):
        idx_v, rows_v, q_v, _, sem = bufs[b]
        base_tok = tok0 + ch * T
        pltpu.sync_copy(idx_hbm.at[pl.ds(base_tok * K, T * K)], idx_v)
        pltpu.async_copy(mn_hbm.at[idx_v], rows_v, sem)
        pltpu.sync_copy(q_hbm.at[pl.ds(base_tok, T)], q_v)

    def finish(ch, b):
        idx_v, rows_v, q_v, out_v, sem = bufs[b]
        pltpu.make_async_copy(mn_hbm.at[idx_v], rows_v, sem).wait()

        def tok_body(t, c2):
            for d in range(ND):
                qv = q_v[t, pl.ds(d * L, L)]
                acc = rows_v[t * K, pl.ds(d * L, L)]
                for k in range(1, K):
                    acc = acc + rows_v[t * K + k, pl.ds(d * L, L)]
                out_v[t, pl.ds(d * L, L)] = acc * qv
            return c2

        lax.fori_loop(0, T, tok_body, 0)
        base_tok = tok0 + ch * T
        pltpu.sync_copy(out_v, out_hbm.at[pl.ds(base_tok, T)])

    issue(0, 0)

    def pair_body(i, carry):
        issue(2 * i + 1, 1)
        finish(2 * i, 0)

        @pl.when(i < CHUNKS // 2 - 1)
        def _():
            issue(2 * i + 2, 0)

        finish(2 * i + 1, 1)
        return carry

    lax.fori_loop(0, CHUNKS // 2, pair_body, 0)


@jax.jit
def _run(idx, q, states):
    means = pl.pallas_call(
        _mean_body,
        grid=(M // BM,),
        in_specs=[pl.BlockSpec((BM, C, D), lambda i: (i, 0, 0))],
        out_specs=pl.BlockSpec((BM, 2 * D), lambda i: (i, 0)),
        out_shape=jax.ShapeDtypeStruct((M, 2 * D), jnp.float32),
    )(states)

    f = functools.partial(
        pl.kernel,
        mesh=plsc.VectorSubcoreMesh(core_axis_name="c", subcore_axis_name="s"),
        out_type=jax.ShapeDtypeStruct((N, D), jnp.float32),
        scratch_types=[
            pltpu.VMEM((T * K,), jnp.int32),
            pltpu.VMEM((T * K,), jnp.int32),
            pltpu.VMEM((T * K, 2 * D), jnp.float32),
            pltpu.VMEM((T * K, 2 * D), jnp.float32),
            pltpu.VMEM((T, D), jnp.float32),
            pltpu.VMEM((T, D), jnp.float32),
            pltpu.VMEM((T, D), jnp.float32),
            pltpu.VMEM((T, D), jnp.float32),
            pltpu.SemaphoreType.DMA,
            pltpu.SemaphoreType.DMA,
        ],
    )(_sc_read)
    return f(idx, q, means)


def kernel(partition_indices, queries, states):
    idx = partition_indices.reshape(N * K).astype(jnp.int32)
    q = queries.reshape(N, D)
    out = _run(idx, q, states)
    return out.reshape(B, S, D)


# speedup vs baseline: 2.5362x; 2.5362x over previous
import functools

import jax
import jax.numpy as jnp
from jax import lax
from jax.experimental import pallas as pl
from jax.experimental.pallas import tpu as pltpu
from jax.experimental.pallas import tpu_sc as plsc

M, C, D = 65536, 8, 64
B, S, K = 8, 2048, 4
L = 16
SV = S // L

NC, NS = 2, 16
NW = NC * NS
DPW = D // NW

BD1, BM1 = 8, 8192


def _mean_body(st_ref, out_ref):
    acc = st_ref[0]
    for c in range(1, C):
        acc = acc + st_ref[c]
    out_ref[...] = acc * (1.0 / C)


def _sc_read(idx_hbm, q_hbm, mn_hbm, out_hbm, mrow_v, idx_v, q_v, out_v):
    wid = lax.axis_index("s") * NC + lax.axis_index("c")

    for j in range(DPW):
        d = wid * DPW + j
        pltpu.sync_copy(mn_hbm.at[d], mrow_v)
        for b in range(B):
            pltpu.sync_copy(idx_hbm.at[pl.ds(b * K * S, K * S)], idx_v)
            pltpu.sync_copy(q_hbm.at[b, d], q_v)

            def svec_body(sv, carry):
                s0 = sv * L
                acc = None
                for k in range(K):
                    iv = idx_v[pl.ds(k * S + s0, L)]
                    g = plsc.load_gather(mrow_v, [iv])
                    acc = g if acc is None else acc + g
                out_v[pl.ds(s0, L)] = acc * q_v[pl.ds(s0, L)]
                return carry

            lax.fori_loop(0, SV, svec_body, 0)
            pltpu.sync_copy(out_v, out_hbm.at[b, d])


@jax.jit
def _run(idx1, q_t, states_t):
    means_t = pl.pallas_call(
        _mean_body,
        grid=(D // BD1, M // BM1),
        in_specs=[pl.BlockSpec((C, BD1, BM1), lambda j, i: (0, j, i))],
        out_specs=pl.BlockSpec((BD1, BM1), lambda j, i: (j, i)),
        out_shape=jax.ShapeDtypeStruct((D, M), jnp.float32),
    )(states_t)

    f = functools.partial(
        pl.kernel,
        mesh=plsc.VectorSubcoreMesh(core_axis_name="c", subcore_axis_name="s"),
        out_type=jax.ShapeDtypeStruct((B, D, S), jnp.float32),
        scratch_types=[
            pltpu.VMEM((M,), jnp.float32),
            pltpu.VMEM((K * S,), jnp.int32),
            pltpu.VMEM((S,), jnp.float32),
            pltpu.VMEM((S,), jnp.float32),
        ],
        compiler_params=pltpu.CompilerParams(needs_layout_passes=False),
    )(_sc_read)
    return f(idx1, q_t, means_t)


def kernel(partition_indices, queries, states):
    states_t = jnp.transpose(states, (1, 2, 0))
    q_t = jnp.transpose(queries, (0, 2, 1))
    idx1 = jnp.transpose(partition_indices, (0, 2, 1)).reshape(B * K * S)
    idx1 = idx1.astype(jnp.int32)
    out_t = _run(idx1, q_t, states_t)
    return jnp.transpose(out_t, (0, 2, 1))


# speedup vs baseline: 3.5370x; 1.3946x over previous
import functools

import jax
import jax.numpy as jnp
from jax import lax
from jax.experimental import pallas as pl
from jax.experimental.pallas import tpu as pltpu
from jax.experimental.pallas import tpu_sc as plsc

M, C, D = 65536, 8, 64
B, S, K = 8, 2048, 4
L = 16
SV = S // L
UNROLL = 4

NC, NS = 2, 16
NW = NC * NS
DPW = D // NW

BD1, BM1 = 16, 8192


def _mean_body(st_ref, out_ref):
    acc = st_ref[0]
    for c in range(1, C):
        acc = acc + st_ref[c]
    out_ref[...] = acc * (1.0 / C)


def _sc_read(idx_hbm, q_hbm, mn_hbm, out_hbm,
             mrow_v, idx0, idx1, q0, q1, o0, o1, sem0, sem1, osem0, osem1):
    wid = lax.axis_index("s") * NC + lax.axis_index("c")
    bufs = ((idx0, q0, o0, sem0, osem0), (idx1, q1, o1, sem1, osem1))

    for j in range(DPW):
        d = wid * DPW + j
        pltpu.sync_copy(mn_hbm.at[d], mrow_v)

        def issue(b):
            idx_v, q_v, _, sem, _ = bufs[b % 2]
            pltpu.async_copy(idx_hbm.at[pl.ds(b * K * S, K * S)], idx_v, sem)
            pltpu.async_copy(q_hbm.at[b, d], q_v, sem)

        issue(0)
        for b in range(B):
            idx_v, q_v, out_v, sem, osem = bufs[b % 2]
            if b + 1 < B:
                issue(b + 1)
            if b >= 2:
                pltpu.make_async_copy(out_v, out_hbm.at[b - 2, d], osem).wait()
            pltpu.make_async_copy(idx_hbm.at[pl.ds(b * K * S, K * S)], idx_v,
                                  sem).wait()
            pltpu.make_async_copy(q_hbm.at[b, d], q_v, sem).wait()

            def svec_body(sv, carry, idx_v=idx_v, q_v=q_v, out_v=out_v):
                base = sv * (L * UNROLL)
                for u in range(UNROLL):
                    s0 = base + u * L
                    acc = None
                    for k in range(K):
                        iv = idx_v[pl.ds(k * S + s0, L)]
                        g = plsc.load_gather(mrow_v, [iv])
                        acc = g if acc is None else acc + g
                    out_v[pl.ds(s0, L)] = acc * q_v[pl.ds(s0, L)]
                return carry

            lax.fori_loop(0, SV // UNROLL, svec_body, 0)
            pltpu.async_copy(out_v, out_hbm.at[b, d], osem)

        for b in (B - 2, B - 1):
            _, _, out_v, _, osem = bufs[b % 2]
            pltpu.make_async_copy(out_v, out_hbm.at[b, d], osem).wait()


@jax.jit
def _run(idx1, q_t, states_t):
    means_t = pl.pallas_call(
        _mean_body,
        grid=(D // BD1, M // BM1),
        in_specs=[pl.BlockSpec((C, BD1, BM1), lambda j, i: (0, j, i))],
        out_specs=pl.BlockSpec((BD1, BM1), lambda j, i: (j, i)),
        out_shape=jax.ShapeDtypeStruct((D, M), jnp.float32),
    )(states_t)

    f = functools.partial(
        pl.kernel,
        mesh=plsc.VectorSubcoreMesh(core_axis_name="c", subcore_axis_name="s"),
        out_type=jax.ShapeDtypeStruct((B, D, S), jnp.float32),
        scratch_types=[
            pltpu.VMEM((M,), jnp.float32),
            pltpu.VMEM((K * S,), jnp.int32),
            pltpu.VMEM((K * S,), jnp.int32),
            pltpu.VMEM((S,), jnp.float32),
            pltpu.VMEM((S,), jnp.float32),
            pltpu.VMEM((S,), jnp.float32),
            pltpu.VMEM((S,), jnp.float32),
            pltpu.SemaphoreType.DMA,
            pltpu.SemaphoreType.DMA,
            pltpu.SemaphoreType.DMA,
            pltpu.SemaphoreType.DMA,
        ],
        compiler_params=pltpu.CompilerParams(needs_layout_passes=False),
    )(_sc_read)
    return f(idx1, q_t, means_t)


def kernel(partition_indices, queries, states):
    states_t = jnp.transpose(states, (1, 2, 0))
    q_t = jnp.transpose(queries, (0, 2, 1))
    idx1 = jnp.transpose(partition_indices, (0, 2, 1)).reshape(B * K * S)
    idx1 = idx1.astype(jnp.int32)
    out_t = _run(idx1, q_t, states_t)
    return jnp.transpose(out_t, (0, 2, 1))


# speedup vs baseline: 3.9040x; 1.1038x over previous
import functools

import jax
import jax.numpy as jnp
from jax import lax
from jax.experimental import pallas as pl
from jax.experimental.pallas import tpu as pltpu
from jax.experimental.pallas import tpu_sc as plsc

M, C, D = 65536, 8, 64
B, S, K = 8, 2048, 4
L = 16
SV = S // L
UNROLL = 8

NC, NS = 2, 16
NW = NC * NS
DPW = D // NW

BD1, BM1 = 16, 16384


def _mean_body(st_ref, out_ref):
    acc = st_ref[0]
    for c in range(1, C):
        acc = acc + st_ref[c]
    out_ref[...] = acc * (1.0 / C)


def _sc_read(idx_hbm, q_hbm, mn_hbm, out_hbm,
             mrow_v, idx0, idx1, q0, q1, o0, o1, sem0, sem1, osem0, osem1):
    wid = lax.axis_index("s") * NC + lax.axis_index("c")
    bufs = ((idx0, q0, o0, sem0, osem0), (idx1, q1, o1, sem1, osem1))

    for j in range(DPW):
        d = wid * DPW + j
        pltpu.sync_copy(mn_hbm.at[d], mrow_v)

        def issue(b):
            idx_v, q_v, _, sem, _ = bufs[b % 2]
            pltpu.async_copy(idx_hbm.at[pl.ds(b * K * S, K * S)], idx_v, sem)
            pltpu.async_copy(q_hbm.at[b, d], q_v, sem)

        issue(0)
        for b in range(B):
            idx_v, q_v, out_v, sem, osem = bufs[b % 2]
            if b + 1 < B:
                issue(b + 1)
            if b >= 2:
                pltpu.make_async_copy(out_v, out_hbm.at[b - 2, d], osem).wait()
            pltpu.make_async_copy(idx_hbm.at[pl.ds(b * K * S, K * S)], idx_v,
                                  sem).wait()
            pltpu.make_async_copy(q_hbm.at[b, d], q_v, sem).wait()

            @plsc.parallel_loop(0, S, step=L, unroll=UNROLL)
            def _svec(s0, idx_v=idx_v, q_v=q_v, out_v=out_v):
                acc = None
                for k in range(K):
                    iv = idx_v[pl.ds(k * S + s0, L)]
                    g = plsc.load_gather(mrow_v, [iv])
                    acc = g if acc is None else acc + g
                out_v[pl.ds(s0, L)] = acc * q_v[pl.ds(s0, L)]
            pltpu.async_copy(out_v, out_hbm.at[b, d], osem)

        for b in (B - 2, B - 1):
            _, _, out_v, _, osem = bufs[b % 2]
            pltpu.make_async_copy(out_v, out_hbm.at[b, d], osem).wait()


@jax.jit
def _run(idx1, q_t, states_t):
    means_t = pl.pallas_call(
        _mean_body,
        grid=(D // BD1, M // BM1),
        in_specs=[pl.BlockSpec((C, BD1, BM1), lambda j, i: (0, j, i))],
        out_specs=pl.BlockSpec((BD1, BM1), lambda j, i: (j, i)),
        out_shape=jax.ShapeDtypeStruct((D, M), jnp.float32),
    )(states_t)

    f = functools.partial(
        pl.kernel,
        mesh=plsc.VectorSubcoreMesh(core_axis_name="c", subcore_axis_name="s"),
        out_type=jax.ShapeDtypeStruct((B, D, S), jnp.float32),
        scratch_types=[
            pltpu.VMEM((M,), jnp.float32),
            pltpu.VMEM((K * S,), jnp.int32),
            pltpu.VMEM((K * S,), jnp.int32),
            pltpu.VMEM((S,), jnp.float32),
            pltpu.VMEM((S,), jnp.float32),
            pltpu.VMEM((S,), jnp.float32),
            pltpu.VMEM((S,), jnp.float32),
            pltpu.SemaphoreType.DMA,
            pltpu.SemaphoreType.DMA,
            pltpu.SemaphoreType.DMA,
            pltpu.SemaphoreType.DMA,
        ],
        compiler_params=pltpu.CompilerParams(needs_layout_passes=False),
    )(_sc_read)
    return f(idx1, q_t, means_t)


def kernel(partition_indices, queries, states):
    states_t = jnp.transpose(states, (1, 2, 0))
    q_t = jnp.transpose(queries, (0, 2, 1))
    idx1 = jnp.transpose(partition_indices, (0, 2, 1)).reshape(B * K * S)
    idx1 = idx1.astype(jnp.int32)
    out_t = _run(idx1, q_t, states_t)
    return jnp.transpose(out_t, (0, 2, 1))
